# Initial kernel scaffold; baseline (speedup 1.0000x reference)
import jax
import jax.numpy as jnp
from jax.experimental import pallas as pl


def kernel(xyz, features, W1, b1, W2, b2):
    raise NotImplementedError("write your pallas kernel here")


# speedup vs baseline: 5.9933x; 5.9933x over previous
import jax
import jax.numpy as jnp
from jax import lax
from jax.experimental import pallas as pl
from jax.experimental.pallas import tpu as pltpu

_NPOINT = 1024
_NSAMPLE = 32
_R2 = 0.2 * 0.2
_TS = 8


def _fps_body(coords_ref, xyz_ref, out_ref):
    _, _, RS, CS = coords_ref.shape
    n = RS * CS
    X = coords_ref[0, 0]
    Y = coords_ref[0, 1]
    Z = coords_ref[0, 2]
    lin = (lax.broadcasted_iota(jnp.int32, (RS, CS), 0) * CS
           + lax.broadcasted_iota(jnp.int32, (RS, CS), 1))
    dists0 = jnp.full((RS, CS), 1e10, jnp.float32)

    def body(i, carry):
        far, dists = carry
        c3 = xyz_ref[0, pl.ds(far, 1), :]
        out_ref[0, pl.ds(i, 1), :] = c3
        dx = X - c3[0:1, 0:1]
        dy = Y - c3[0:1, 1:2]
        dz = Z - c3[0:1, 2:3]
        d = dx * dx + dy * dy + dz * dz
        dists = jnp.minimum(dists, d)
        m = jnp.max(dists)
        cand = jnp.where(dists == m, lin, jnp.int32(n))
        far = jnp.min(cand)
        return far, dists

    lax.fori_loop(0, _NPOINT, body, (jnp.int32(0), dists0))


def _cumsum_lanes(x):
    r, n = x.shape
    sh = 1
    while sh < n:
        pad = jnp.zeros((r, sh), x.dtype)
        x = x + jnp.concatenate([pad, x[:, :n - sh]], axis=1)
        sh *= 2
    return x


def _group_body(xyzT_ref, xyz_ref, featT_ref, nxyz_ref, W1a_ref, W1b_ref,
                b1_ref, W2_ref, b2_ref, out_ref, A_ref):
    t = pl.program_id(1)
    n = xyz_ref.shape[1]
    K = _NSAMPLE

    @pl.when(t == 0)
    def _():
        A_ref[...] = (
            jnp.dot(xyz_ref[0], W1a_ref[...],
                    preferred_element_type=jnp.float32)
            + jnp.dot(featT_ref[0], W1b_ref[...],
                      preferred_element_type=jnp.float32)
            + b1_ref[0:1, :])

    cz = nxyz_ref[0]
    X = xyzT_ref[0, 0:1, :]
    Y = xyzT_ref[0, 1:2, :]
    Z = xyzT_ref[0, 2:3, :]
    dx = cz[:, 0:1] - X
    dy = cz[:, 1:2] - Y
    dz = cz[:, 2:3] - Z
    d = dx * dx + dy * dy + dz * dz
    mask = d <= _R2
    rk = _cumsum_lanes(mask.astype(jnp.int32))
    slot = rk - 1
    kio = lax.broadcasted_iota(jnp.int32, (_TS, K, n), 1)
    P = jnp.where((slot[:, None, :] == kio) & mask[:, None, :], 1.0, 0.0)
    G = jnp.dot(P.reshape(_TS * K, n), A_ref[...],
                preferred_element_type=jnp.float32)
    q = jnp.dot(cz, W1a_ref[...], preferred_element_type=jnp.float32)
    h1 = jnp.maximum(G.reshape(_TS, K, 32) - q[:, None, :], 0.0)
    h2 = jnp.dot(h1.reshape(_TS * K, 32), W2_ref[...],
                 preferred_element_type=jnp.float32) + b2_ref[0:1, :]
    h2 = jnp.maximum(h2, 0.0).reshape(_TS, K, 64)
    cnt = rk[:, n - 1:n]
    valid = lax.broadcasted_iota(jnp.int32, (_TS, K), 1) < cnt
    h2 = h2 * valid.astype(jnp.float32)[:, :, None]
    out_ref[0] = jnp.max(h2, axis=1)


def kernel(xyz, features, W1, b1, W2, b2):
    B, N, _ = xyz.shape
    S = _NPOINT
    RS = 8
    CS = N // RS

    xyzT = jnp.transpose(xyz, (0, 2, 1))
    coords = xyzT.reshape(B, 3, RS, CS)
    featT = jnp.transpose(features, (0, 2, 1))
    C = featT.shape[2]

    new_xyz = pl.pallas_call(
        _fps_body,
        grid=(B,),
        in_specs=[
            pl.BlockSpec((1, 3, RS, CS), lambda b: (b, 0, 0, 0)),
            pl.BlockSpec((1, N, 3), lambda b: (b, 0, 0)),
        ],
        out_specs=pl.BlockSpec((1, S, 3), lambda b: (b, 0, 0)),
        out_shape=jax.ShapeDtypeStruct((B, S, 3), jnp.float32),
    )(coords, xyz)

    NT = S // _TS
    newf = pl.pallas_call(
        _group_body,
        grid=(B, NT),
        in_specs=[
            pl.BlockSpec((1, 3, N), lambda b, t: (b, 0, 0)),
            pl.BlockSpec((1, N, 3), lambda b, t: (b, 0, 0)),
            pl.BlockSpec((1, N, C), lambda b, t: (b, 0, 0)),
            pl.BlockSpec((1, _TS, 3), lambda b, t: (b, t, 0)),
            pl.BlockSpec((3, 32), lambda b, t: (0, 0)),
            pl.BlockSpec((16, 32), lambda b, t: (0, 0)),
            pl.BlockSpec((1, 32), lambda b, t: (0, 0)),
            pl.BlockSpec((32, 64), lambda b, t: (0, 0)),
            pl.BlockSpec((1, 64), lambda b, t: (0, 0)),
        ],
        out_specs=pl.BlockSpec((1, _TS, 64), lambda b, t: (b, t, 0)),
        out_shape=jax.ShapeDtypeStruct((B, S, 64), jnp.float32),
        scratch_shapes=[pltpu.VMEM((N, 32), jnp.float32)],
    )(xyzT, xyz, featT, new_xyz, W1[0:3], W1[3:19],
      b1.reshape(1, 32), W2, b2.reshape(1, 64))

    new_features = jnp.transpose(newf, (0, 2, 1))
    return (new_xyz, new_features)
